# Initial kernel scaffold; baseline (speedup 1.0000x reference)
import functools

import jax
import jax.numpy as jnp
from jax.experimental import pallas as pl
from jax.experimental.pallas import tpu as pltpu

N_EMBD = 512
N_EXPERTS = 32
D_FF = 4 * N_EMBD
TOKENS = 8192
CAPACITY = 128


def _router_kernel(x_ref, wg_ref, idx_ref, gate_ref):
    x = x_ref[...]
    wg = wg_ref[...]
    logits = jnp.dot(x, wg, preferred_element_type=jnp.float32)
    m = jnp.max(logits, axis=1, keepdims=True)
    ex = jnp.exp(logits - m)
    probs = ex / jnp.sum(ex, axis=1, keepdims=True)
    pt = probs.T

    tio = jax.lax.broadcasted_iota(jnp.int32, (N_EXPERTS, TOKENS), 1)
    cio = jax.lax.broadcasted_iota(jnp.int32, (N_EXPERTS, CAPACITY), 1)

    def body(c, carry):
        ptc, acc_i, acc_g = carry
        mx = jnp.max(ptc, axis=1, keepdims=True)
        cand = jnp.where(ptc == mx, tio, TOKENS)
        win = jnp.min(cand, axis=1, keepdims=True)
        ptc = jnp.where(tio == win, -1.0, ptc)
        acc_i = jnp.where(cio == c, win, acc_i)
        acc_g = jnp.where(cio == c, mx, acc_g)
        return ptc, acc_i, acc_g

    _, acc_i, acc_g = jax.lax.fori_loop(
        0, CAPACITY, body,
        (pt,
         jnp.zeros((N_EXPERTS, CAPACITY), jnp.int32),
         jnp.zeros((N_EXPERTS, CAPACITY), jnp.float32)))
    idx_ref[...] = acc_i
    gate_ref[...] = acc_g


def _ffn_kernel(idx_sm, x_ref, w1_ref, b1_ref, w2_ref, b2_ref, gate_ref,
                y_ref, xe_ref, ye_ref):
    e = pl.program_id(0)

    @pl.when(e == 0)
    def _():
        y_ref[...] = jnp.zeros_like(y_ref)

    def gather_body(c, _):
        t = idx_sm[e * CAPACITY + c]
        xe_ref[pl.ds(c, 1), :] = x_ref[pl.ds(t, 1), :]
        return 0

    jax.lax.fori_loop(0, CAPACITY, gather_body, 0)

    xe = xe_ref[...]
    h = jnp.dot(xe, w1_ref[0], preferred_element_type=jnp.float32)
    h = jax.nn.gelu(h + b1_ref[0])
    ye = jnp.dot(h, w2_ref[0], preferred_element_type=jnp.float32)
    ye = ye + b2_ref[0]
    g = gate_ref[0].reshape(CAPACITY, 1)
    ye_ref[...] = ye * g

    def scatter_body(c, _):
        t = idx_sm[e * CAPACITY + c]
        y_ref[pl.ds(t, 1), :] += ye_ref[pl.ds(c, 1), :]
        return 0

    jax.lax.fori_loop(0, CAPACITY, scatter_body, 0)


@jax.jit
def kernel(x, Wg, W1, b1, W2, b2):
    idx, gates = pl.pallas_call(
        _router_kernel,
        out_shape=(
            jax.ShapeDtypeStruct((N_EXPERTS, CAPACITY), jnp.int32),
            jax.ShapeDtypeStruct((N_EXPERTS, CAPACITY), jnp.float32),
        ),
    )(x, Wg)

    grid_spec = pltpu.PrefetchScalarGridSpec(
        num_scalar_prefetch=1,
        grid=(N_EXPERTS,),
        in_specs=[
            pl.BlockSpec((TOKENS, N_EMBD), lambda e, i: (0, 0)),
            pl.BlockSpec((1, N_EMBD, D_FF), lambda e, i: (e, 0, 0)),
            pl.BlockSpec((1, D_FF), lambda e, i: (e, 0)),
            pl.BlockSpec((1, D_FF, N_EMBD), lambda e, i: (e, 0, 0)),
            pl.BlockSpec((1, N_EMBD), lambda e, i: (e, 0)),
            pl.BlockSpec((1, CAPACITY), lambda e, i: (e, 0)),
        ],
        out_specs=pl.BlockSpec((TOKENS, N_EMBD), lambda e, i: (0, 0)),
        scratch_shapes=[
            pltpu.VMEM((CAPACITY, N_EMBD), jnp.float32),
            pltpu.VMEM((CAPACITY, N_EMBD), jnp.float32),
        ],
    )

    y = pl.pallas_call(
        _ffn_kernel,
        grid_spec=grid_spec,
        out_shape=jax.ShapeDtypeStruct((TOKENS, N_EMBD), jnp.float32),
        compiler_params=pltpu.CompilerParams(
            dimension_semantics=("arbitrary",),
        ),
    )(idx.reshape(-1), x, W1, b1, W2, b2, gates)
    return y


# speedup vs baseline: 1.6689x; 1.6689x over previous
import functools

import jax
import jax.numpy as jnp
from jax.experimental import pallas as pl
from jax.experimental.pallas import tpu as pltpu

N_EMBD = 512
N_EXPERTS = 32
D_FF = 4 * N_EMBD
TOKENS = 8192
CAPACITY = 128


def _router_kernel(x_ref, wg_ref, idx_ref, gate_ref):
    x = x_ref[...]
    wg = wg_ref[...]
    logits = jnp.dot(x, wg, preferred_element_type=jnp.float32)
    m = jnp.max(logits, axis=1, keepdims=True)
    ex = jnp.exp(logits - m)
    probs = ex / jnp.sum(ex, axis=1, keepdims=True)
    pt = probs.T

    tio = jax.lax.broadcasted_iota(jnp.int32, (N_EXPERTS, TOKENS), 1)
    cio = jax.lax.broadcasted_iota(jnp.int32, (N_EXPERTS, CAPACITY), 1)

    def body(c, carry):
        ptc, acc_i, acc_g = carry
        mx = jnp.max(ptc, axis=1, keepdims=True)
        cand = jnp.where(ptc == mx, tio, TOKENS)
        win = jnp.min(cand, axis=1, keepdims=True)
        ptc = jnp.where(tio == win, -1.0, ptc)
        acc_i = jnp.where(cio == c, win, acc_i)
        acc_g = jnp.where(cio == c, mx, acc_g)
        return ptc, acc_i, acc_g

    _, acc_i, acc_g = jax.lax.fori_loop(
        0, CAPACITY, body,
        (pt,
         jnp.zeros((N_EXPERTS, CAPACITY), jnp.int32),
         jnp.zeros((N_EXPERTS, CAPACITY), jnp.float32)))
    idx_ref[...] = acc_i
    gate_ref[...] = acc_g


def _ffn_kernel(idx_sm, x_ref, w1_ref, b1_ref, w2_ref, b2_ref, gate_ref,
                y_ref, xe_ref, ye_ref):
    e = pl.program_id(0)

    @pl.when(e == 0)
    def _():
        y_ref[...] = jnp.zeros_like(y_ref)

    def gather_body(c, _):
        t = idx_sm[e * CAPACITY + c]
        xe_ref[pl.ds(c, 1), :] = x_ref[pl.ds(t, 1), :]
        return 0

    jax.lax.fori_loop(0, CAPACITY, gather_body, 0)

    xe = xe_ref[...]
    h = jnp.dot(xe, w1_ref[0], preferred_element_type=jnp.float32)
    h = jax.nn.gelu(h + b1_ref[0])
    ye = jnp.dot(h, w2_ref[0], preferred_element_type=jnp.float32)
    ye = ye + b2_ref[0]
    g = gate_ref[0, 0].reshape(CAPACITY, 1)
    ye_ref[...] = ye * g

    def scatter_body(c, _):
        t = idx_sm[e * CAPACITY + c]
        y_ref[pl.ds(t, 1), :] += ye_ref[pl.ds(c, 1), :]
        return 0

    jax.lax.fori_loop(0, CAPACITY, scatter_body, 0)


@jax.jit
def kernel(x, Wg, W1, b1, W2, b2):
    idx, gates = pl.pallas_call(
        _router_kernel,
        out_shape=(
            jax.ShapeDtypeStruct((N_EXPERTS, CAPACITY), jnp.int32),
            jax.ShapeDtypeStruct((N_EXPERTS, CAPACITY), jnp.float32),
        ),
    )(x, Wg)

    grid_spec = pltpu.PrefetchScalarGridSpec(
        num_scalar_prefetch=1,
        grid=(N_EXPERTS,),
        in_specs=[
            pl.BlockSpec((TOKENS, N_EMBD), lambda e, i: (0, 0)),
            pl.BlockSpec((1, N_EMBD, D_FF), lambda e, i: (e, 0, 0)),
            pl.BlockSpec((1, 1, D_FF), lambda e, i: (e, 0, 0)),
            pl.BlockSpec((1, D_FF, N_EMBD), lambda e, i: (e, 0, 0)),
            pl.BlockSpec((1, 1, N_EMBD), lambda e, i: (e, 0, 0)),
            pl.BlockSpec((1, 1, CAPACITY), lambda e, i: (e, 0, 0)),
        ],
        out_specs=pl.BlockSpec((TOKENS, N_EMBD), lambda e, i: (0, 0)),
        scratch_shapes=[
            pltpu.VMEM((CAPACITY, N_EMBD), jnp.float32),
            pltpu.VMEM((CAPACITY, N_EMBD), jnp.float32),
        ],
    )

    y = pl.pallas_call(
        _ffn_kernel,
        grid_spec=grid_spec,
        out_shape=jax.ShapeDtypeStruct((TOKENS, N_EMBD), jnp.float32),
        compiler_params=pltpu.CompilerParams(
            dimension_semantics=("arbitrary",),
        ),
    )(idx.reshape(-1), x, W1, b1.reshape(N_EXPERTS, 1, D_FF), W2,
      b2.reshape(N_EXPERTS, 1, N_EMBD), gates.reshape(N_EXPERTS, 1, CAPACITY))
    return y
